# baseline (speedup 1.0000x reference)
import functools

import jax
import jax.numpy as jnp
from jax import lax
from jax.experimental import pallas as pl
from jax.experimental.pallas import tpu as pltpu
from jax.experimental.pallas import tpu_sc as plsc

N = 10000
D = 512
H = 8
NC = 2
NS = 16
LN = 16
NW = NC * NS
CH = 16
NCH = N // CH
MAXC = -(-NCH // NW)
SLAB = H * MAXC * CH
KD = D // LN


def _worker_id():
    return lax.axis_index("s") * NC + lax.axis_index("c")


def _span(wid):
    c0 = (wid * NCH) // NW
    c1 = ((wid + 1) * NCH) // NW
    return c0, c1 - c0


def _sc_logits(x_ref, w2_ref, w1_ref, b_ref, logit_ref, stat_ref,
               w2_v, w1_v, r0_v, b_v, rb0, rb1, pb_v, lb_v, st_v,
               sem0, sem1):
    wid = _worker_id()
    c0, n = _span(wid)
    iota = lax.broadcasted_iota(jnp.int32, (LN,), 0)

    pltpu.sync_copy(w2_ref, w2_v)
    pltpu.sync_copy(w1_ref, w1_v)
    pltpu.sync_copy(b_ref, b_v)
    pltpu.sync_copy(x_ref.at[pl.ds(0, D)], r0_v)
    b16 = b_v[pl.ds(0, LN)]

    cvals = []
    for h in range(H):
        acc = jnp.zeros((LN,), jnp.float32)
        for k in range(KD):
            acc = acc + w1_v[pl.ds(h * D + k * LN, LN)] * r0_v[pl.ds(k * LN, LN)]
        cvals.append(jnp.sum(acc) + b16[h])

    def row_copy(c, buf, sem):
        return pltpu.make_async_copy(
            x_ref.at[pl.ds((c0 + c) * CH * D, CH * D)], buf, sem)

    def process(buf, i):
        for hg in range(2):
            for kk in range(4):
                w2r = [[w2_v[pl.ds((hg * 4 + h) * D + (kk * 8 + k) * LN, LN)]
                        for k in range(8)] for h in range(4)]

                def jbody(j, _, hg=hg, kk=kk, w2r=w2r):
                    accs = [jnp.zeros((LN,), jnp.float32) for _ in range(4)]
                    for k in range(8):
                        v = buf[pl.ds(j * D + (kk * 8 + k) * LN, LN)]
                        for h in range(4):
                            accs[h] = accs[h] + v * w2r[h][k]
                    for h in range(4):
                        idx = (hg * 4 + h) * (CH * LN) + j * LN
                        if kk == 0:
                            pb_v[pl.ds(idx, LN)] = accs[h]
                        else:
                            plsc.addupdate(pb_v.at[pl.ds(idx, LN)], accs[h])
                    return 0

                lax.fori_loop(0, CH, jbody, 0)
        for h in range(H):
            def abody(j, lv, h=h):
                s = jnp.sum(pb_v[pl.ds(h * CH * LN + j * LN, LN)])
                return jnp.where(iota == j, jnp.full((LN,), s), lv)
            lv = lax.fori_loop(0, CH, abody, jnp.zeros((LN,), jnp.float32))
            lb_v[pl.ds(h * (MAXC * CH) + i * CH, CH)] = lv

    row_copy(0, rb0, sem0).start()

    def cbody(i, _):
        slot = lax.rem(i, 2)

        @pl.when(slot == 0)
        def _():
            @pl.when(i + 1 < n)
            def _():
                row_copy(i + 1, rb1, sem1).start()
            row_copy(i, rb0, sem0).wait()
            process(rb0, i)

        @pl.when(slot == 1)
        def _():
            @pl.when(i + 1 < n)
            def _():
                row_copy(i + 1, rb0, sem0).start()
            row_copy(i, rb1, sem1).wait()
            process(rb1, i)

        return 0

    lax.fori_loop(0, n, cbody, 0)

    m16 = jnp.full((LN,), -jnp.inf, jnp.float32)
    s16 = jnp.zeros((LN,), jnp.float32)
    for h in range(H):
        cb = jnp.full((LN,), cvals[h])

        def s1(i2, mv, h=h, cb=cb):
            off = h * (MAXC * CH) + i2 * CH
            v = lb_v[pl.ds(off, CH)] + cb
            v = jnp.where(v >= 0.0, v, 0.01 * v)
            lb_v[pl.ds(off, CH)] = v
            return jnp.maximum(mv, v)

        mv = lax.fori_loop(0, n, s1, jnp.full((LN,), -jnp.inf, jnp.float32))
        mh = jnp.max(mv)
        mb = jnp.full((LN,), mh)

        def s2(i2, sv, h=h, mb=mb):
            v = lb_v[pl.ds(h * (MAXC * CH) + i2 * CH, CH)]
            return sv + jnp.exp(v - mb)

        sv = lax.fori_loop(0, n, s2, jnp.zeros((LN,), jnp.float32))
        m16 = jnp.where(iota == h, mb, m16)
        s16 = jnp.where(iota == h, jnp.full((LN,), jnp.sum(sv)), s16)

    st_v[pl.ds(0, LN)] = m16
    st_v[pl.ds(LN, LN)] = s16
    pltpu.sync_copy(lb_v, logit_ref.at[wid])
    pltpu.sync_copy(st_v.at[pl.ds(0, LN)],
                    stat_ref.at[pl.ds(wid * LN, LN)])
    pltpu.sync_copy(st_v.at[pl.ds(LN, LN)],
                    stat_ref.at[pl.ds(NW * LN + wid * LN, LN)])


def _sc_accum(x_ref, logit_ref, stat_ref, part_ref,
              lb_v, sb_v, ab_v, rb0, rb1, sem0, sem1):
    wid = _worker_id()
    c0, n = _span(wid)

    pltpu.sync_copy(logit_ref.at[wid], lb_v)
    pltpu.sync_copy(stat_ref, sb_v)

    def mb_(w, mv):
        return jnp.maximum(mv, sb_v[pl.ds(w * LN, LN)])
    m16 = lax.fori_loop(0, NW, mb_, jnp.full((LN,), -jnp.inf, jnp.float32))

    def sb_(w, sv):
        mw = sb_v[pl.ds(w * LN, LN)]
        sw = sb_v[pl.ds(NW * LN + w * LN, LN)]
        return sv + sw * jnp.exp(mw - m16)
    s16 = lax.fori_loop(0, NW, sb_, jnp.zeros((LN,), jnp.float32))

    i16 = jnp.float32(1.0) / s16
    mbs = [jnp.full((LN,), m16[h]) for h in range(H)]
    ibs = [jnp.full((LN,), i16[h]) for h in range(H)]

    for k in range(KD):
        ab_v[pl.ds(k * LN, LN)] = jnp.zeros((LN,), jnp.float32)

    def row_copy(c, buf, sem):
        return pltpu.make_async_copy(
            x_ref.at[pl.ds((c0 + c) * CH * D, CH * D)], buf, sem)

    def process(buf, i):
        qv = jnp.zeros((LN,), jnp.float32)
        for h in range(H):
            lv = lb_v[pl.ds(h * (MAXC * CH) + i * CH, CH)]
            qv = qv + jnp.exp(lv - mbs[h]) * ibs[h]
        qbs = [jnp.full((LN,), qv[j]) for j in range(CH)]
        for k in range(KD):
            a = ab_v[pl.ds(k * LN, LN)]
            for j in range(CH):
                a = a + qbs[j] * buf[pl.ds(j * D + k * LN, LN)]
            ab_v[pl.ds(k * LN, LN)] = a

    row_copy(0, rb0, sem0).start()

    def cbody(i, _):
        slot = lax.rem(i, 2)

        @pl.when(slot == 0)
        def _():
            @pl.when(i + 1 < n)
            def _():
                row_copy(i + 1, rb1, sem1).start()
            row_copy(i, rb0, sem0).wait()
            process(rb0, i)

        @pl.when(slot == 1)
        def _():
            @pl.when(i + 1 < n)
            def _():
                row_copy(i + 1, rb0, sem0).start()
            row_copy(i, rb1, sem1).wait()
            process(rb1, i)

        return 0

    lax.fori_loop(0, n, cbody, 0)
    pltpu.sync_copy(ab_v, part_ref.at[wid])


def _sc_merge(part_ref, out_ref, buf_v, ob_v):
    wid = _worker_id()
    pltpu.sync_copy(part_ref, buf_v)
    col = wid * LN
    acc = jnp.zeros((LN,), jnp.float32)
    for r in range(NW):
        acc = acc + buf_v[pl.ds(r * D + col, LN)]
    ob_v[pl.ds(0, LN)] = jnp.maximum(acc * jnp.float32(1.0 / H), 0.0)
    pltpu.sync_copy(ob_v, out_ref.at[pl.ds(col, LN)])


def _mesh():
    return plsc.VectorSubcoreMesh(core_axis_name="c", subcore_axis_name="s")


_SC_PARAMS = pltpu.CompilerParams(needs_layout_passes=False)


@jax.jit
def _run(attention_mx, W, b):
    x_flat = attention_mx.reshape(N * D)
    w2 = W[:, D:].reshape(H * D)
    w1 = W[:, :D].reshape(H * D)
    b16 = jnp.pad(b, (0, LN - H))

    logits, stats = pl.kernel(
        _sc_logits,
        out_type=(jax.ShapeDtypeStruct((NW, SLAB), jnp.float32),
                  jax.ShapeDtypeStruct((2 * NW * LN,), jnp.float32)),
        mesh=_mesh(),
        compiler_params=_SC_PARAMS,
        scratch_types=[
            pltpu.VMEM((H * D,), jnp.float32),
            pltpu.VMEM((H * D,), jnp.float32),
            pltpu.VMEM((D,), jnp.float32),
            pltpu.VMEM((LN,), jnp.float32),
            pltpu.VMEM((CH * D,), jnp.float32),
            pltpu.VMEM((CH * D,), jnp.float32),
            pltpu.VMEM((H * CH * LN,), jnp.float32),
            pltpu.VMEM((SLAB,), jnp.float32),
            pltpu.VMEM((2 * LN,), jnp.float32),
            pltpu.SemaphoreType.DMA,
            pltpu.SemaphoreType.DMA,
        ],
    )(x_flat, w2, w1, b16)

    parts = pl.kernel(
        _sc_accum,
        out_type=jax.ShapeDtypeStruct((NW, D), jnp.float32),
        mesh=_mesh(),
        compiler_params=_SC_PARAMS,
        scratch_types=[
            pltpu.VMEM((SLAB,), jnp.float32),
            pltpu.VMEM((2 * NW * LN,), jnp.float32),
            pltpu.VMEM((D,), jnp.float32),
            pltpu.VMEM((CH * D,), jnp.float32),
            pltpu.VMEM((CH * D,), jnp.float32),
            pltpu.SemaphoreType.DMA,
            pltpu.SemaphoreType.DMA,
        ],
    )(x_flat, logits, stats)

    parts = parts.reshape(NW * D)
    out = pl.kernel(
        _sc_merge,
        out_type=jax.ShapeDtypeStruct((D,), jnp.float32),
        mesh=_mesh(),
        compiler_params=_SC_PARAMS,
        scratch_types=[
            pltpu.VMEM((NW * D,), jnp.float32),
            pltpu.VMEM((LN,), jnp.float32),
        ],
    )(parts)

    return out


def kernel(attention_mx, W, b):
    return _run(attention_mx, W, b)


# speedup vs baseline: 1.0092x; 1.0092x over previous
import functools

import jax
import jax.numpy as jnp
from jax import lax
from jax.experimental import pallas as pl
from jax.experimental.pallas import tpu as pltpu
from jax.experimental.pallas import tpu_sc as plsc

N = 10000
D = 512
H = 8
NC = 2
NS = 16
LN = 16
NW = NC * NS
CH = 16
NCH = N // CH
MAXC = -(-NCH // NW)
SLAB = H * MAXC * CH
KD = D // LN


def _worker_id():
    return lax.axis_index("s") * NC + lax.axis_index("c")


def _span(wid):
    c0 = (wid * NCH) // NW
    c1 = ((wid + 1) * NCH) // NW
    return c0, c1 - c0


def _sc_logits(x_ref, w2_ref, w1_ref, b_ref, logit_ref, stat_ref,
               w2_v, w1_v, r0_v, b_v, rb0, rb1, pb_v, lb_v, st_v,
               sem0, sem1):
    wid = _worker_id()
    c0, n = _span(wid)
    iota = lax.broadcasted_iota(jnp.int32, (LN,), 0)

    pltpu.sync_copy(w2_ref, w2_v)
    pltpu.sync_copy(w1_ref, w1_v)
    pltpu.sync_copy(b_ref, b_v)
    pltpu.sync_copy(x_ref.at[pl.ds(0, D)], r0_v)
    b16 = b_v[pl.ds(0, LN)]

    cvals = []
    for h in range(H):
        acc = jnp.zeros((LN,), jnp.float32)
        for k in range(KD):
            acc = acc + w1_v[pl.ds(h * D + k * LN, LN)] * r0_v[pl.ds(k * LN, LN)]
        cvals.append(jnp.sum(acc) + b16[h])

    def row_copy(c, buf, sem):
        return pltpu.make_async_copy(
            x_ref.at[pl.ds((c0 + c) * CH * D, CH * D)], buf, sem)

    def process(buf, i):
        for hg in range(2):
            for kk in range(4):
                w2r = [[w2_v[pl.ds((hg * 4 + h) * D + (kk * 8 + k) * LN, LN)]
                        for k in range(8)] for h in range(4)]

                def jbody(jj, _, hg=hg, kk=kk, w2r=w2r):
                    accs = [[jnp.zeros((LN,), jnp.float32) for _ in range(4)]
                            for _ in range(2)]
                    for k in range(8):
                        for r in range(2):
                            v = buf[pl.ds((2 * jj + r) * D
                                          + (kk * 8 + k) * LN, LN)]
                            for h in range(4):
                                accs[r][h] = accs[r][h] + v * w2r[h][k]
                    for r in range(2):
                        for h in range(4):
                            idx = ((hg * 4 + h) * (CH * LN)
                                   + (2 * jj + r) * LN)
                            if kk == 0:
                                pb_v[pl.ds(idx, LN)] = accs[r][h]
                            else:
                                plsc.addupdate(pb_v.at[pl.ds(idx, LN)],
                                               accs[r][h])
                    return 0

                lax.fori_loop(0, CH // 2, jbody, 0)
        for h in range(H):
            def abody(jj, lv, h=h):
                lv0, lv1 = lv
                s0 = jnp.sum(pb_v[pl.ds(h * CH * LN + (2 * jj) * LN, LN)])
                s1 = jnp.sum(pb_v[pl.ds(h * CH * LN + (2 * jj + 1) * LN, LN)])
                lv0 = jnp.where(iota == 2 * jj, jnp.full((LN,), s0), lv0)
                lv1 = jnp.where(iota == 2 * jj + 1, jnp.full((LN,), s1), lv1)
                return (lv0, lv1)
            z = jnp.zeros((LN,), jnp.float32)
            lv0, lv1 = lax.fori_loop(0, CH // 2, abody, (z, z))
            lb_v[pl.ds(h * (MAXC * CH) + i * CH, CH)] = lv0 + lv1

    row_copy(0, rb0, sem0).start()

    def cbody(i, _):
        slot = lax.rem(i, 2)

        @pl.when(slot == 0)
        def _():
            @pl.when(i + 1 < n)
            def _():
                row_copy(i + 1, rb1, sem1).start()
            row_copy(i, rb0, sem0).wait()
            process(rb0, i)

        @pl.when(slot == 1)
        def _():
            @pl.when(i + 1 < n)
            def _():
                row_copy(i + 1, rb0, sem0).start()
            row_copy(i, rb1, sem1).wait()
            process(rb1, i)

        return 0

    lax.fori_loop(0, n, cbody, 0)

    m16 = jnp.full((LN,), -jnp.inf, jnp.float32)
    s16 = jnp.zeros((LN,), jnp.float32)
    for h in range(H):
        cb = jnp.full((LN,), cvals[h])

        def s1(i2, mv, h=h, cb=cb):
            off = h * (MAXC * CH) + i2 * CH
            v = lb_v[pl.ds(off, CH)] + cb
            v = jnp.where(v >= 0.0, v, 0.01 * v)
            lb_v[pl.ds(off, CH)] = v
            return jnp.maximum(mv, v)

        mv = lax.fori_loop(0, n, s1, jnp.full((LN,), -jnp.inf, jnp.float32))
        mh = jnp.max(mv)
        mb = jnp.full((LN,), mh)

        def s2(i2, sv, h=h, mb=mb):
            v = lb_v[pl.ds(h * (MAXC * CH) + i2 * CH, CH)]
            return sv + jnp.exp(v - mb)

        sv = lax.fori_loop(0, n, s2, jnp.zeros((LN,), jnp.float32))
        m16 = jnp.where(iota == h, mb, m16)
        s16 = jnp.where(iota == h, jnp.full((LN,), jnp.sum(sv)), s16)

    st_v[pl.ds(0, LN)] = m16
    st_v[pl.ds(LN, LN)] = s16
    pltpu.sync_copy(lb_v, logit_ref.at[wid])
    pltpu.sync_copy(st_v.at[pl.ds(0, LN)],
                    stat_ref.at[pl.ds(wid * LN, LN)])
    pltpu.sync_copy(st_v.at[pl.ds(LN, LN)],
                    stat_ref.at[pl.ds(NW * LN + wid * LN, LN)])


def _sc_accum(x_ref, logit_ref, stat_ref, part_ref,
              lb_v, sb_v, ab_v, rb0, rb1, sem0, sem1):
    wid = _worker_id()
    c0, n = _span(wid)

    pltpu.sync_copy(logit_ref.at[wid], lb_v)
    pltpu.sync_copy(stat_ref, sb_v)

    def mb_(w, mv):
        return jnp.maximum(mv, sb_v[pl.ds(w * LN, LN)])
    m16 = lax.fori_loop(0, NW, mb_, jnp.full((LN,), -jnp.inf, jnp.float32))

    def sb_(w, sv):
        mw = sb_v[pl.ds(w * LN, LN)]
        sw = sb_v[pl.ds(NW * LN + w * LN, LN)]
        return sv + sw * jnp.exp(mw - m16)
    s16 = lax.fori_loop(0, NW, sb_, jnp.zeros((LN,), jnp.float32))

    i16 = jnp.float32(1.0) / s16
    mbs = [jnp.full((LN,), m16[h]) for h in range(H)]
    ibs = [jnp.full((LN,), i16[h]) for h in range(H)]

    for k in range(KD):
        ab_v[pl.ds(k * LN, LN)] = jnp.zeros((LN,), jnp.float32)

    def row_copy(c, buf, sem):
        return pltpu.make_async_copy(
            x_ref.at[pl.ds((c0 + c) * CH * D, CH * D)], buf, sem)

    def process(buf, i):
        qv = jnp.zeros((LN,), jnp.float32)
        for h in range(H):
            lv = lb_v[pl.ds(h * (MAXC * CH) + i * CH, CH)]
            qv = qv + jnp.exp(lv - mbs[h]) * ibs[h]
        qbs = [jnp.full((LN,), qv[j]) for j in range(CH)]
        for k in range(KD):
            parts = [ab_v[pl.ds(k * LN, LN)]] + [
                jnp.zeros((LN,), jnp.float32) for _ in range(3)]
            for j in range(CH):
                parts[j % 4] = parts[j % 4] + qbs[j] * buf[
                    pl.ds(j * D + k * LN, LN)]
            ab_v[pl.ds(k * LN, LN)] = (
                (parts[0] + parts[1]) + (parts[2] + parts[3]))

    row_copy(0, rb0, sem0).start()

    def cbody(i, _):
        slot = lax.rem(i, 2)

        @pl.when(slot == 0)
        def _():
            @pl.when(i + 1 < n)
            def _():
                row_copy(i + 1, rb1, sem1).start()
            row_copy(i, rb0, sem0).wait()
            process(rb0, i)

        @pl.when(slot == 1)
        def _():
            @pl.when(i + 1 < n)
            def _():
                row_copy(i + 1, rb0, sem0).start()
            row_copy(i, rb1, sem1).wait()
            process(rb1, i)

        return 0

    lax.fori_loop(0, n, cbody, 0)
    pltpu.sync_copy(ab_v, part_ref.at[wid])


def _sc_merge(part_ref, out_ref, buf_v, ob_v):
    wid = _worker_id()
    pltpu.sync_copy(part_ref, buf_v)
    col = wid * LN
    acc = jnp.zeros((LN,), jnp.float32)
    for r in range(NW):
        acc = acc + buf_v[pl.ds(r * D + col, LN)]
    ob_v[pl.ds(0, LN)] = jnp.maximum(acc * jnp.float32(1.0 / H), 0.0)
    pltpu.sync_copy(ob_v, out_ref.at[pl.ds(col, LN)])


def _mesh():
    return plsc.VectorSubcoreMesh(core_axis_name="c", subcore_axis_name="s")


_SC_PARAMS = pltpu.CompilerParams(needs_layout_passes=False)


@jax.jit
def _run(attention_mx, W, b):
    x_flat = attention_mx.reshape(N * D)
    w2 = W[:, D:].reshape(H * D)
    w1 = W[:, :D].reshape(H * D)
    b16 = jnp.pad(b, (0, LN - H))

    logits, stats = pl.kernel(
        _sc_logits,
        out_type=(jax.ShapeDtypeStruct((NW, SLAB), jnp.float32),
                  jax.ShapeDtypeStruct((2 * NW * LN,), jnp.float32)),
        mesh=_mesh(),
        compiler_params=_SC_PARAMS,
        scratch_types=[
            pltpu.VMEM((H * D,), jnp.float32),
            pltpu.VMEM((H * D,), jnp.float32),
            pltpu.VMEM((D,), jnp.float32),
            pltpu.VMEM((LN,), jnp.float32),
            pltpu.VMEM((CH * D,), jnp.float32),
            pltpu.VMEM((CH * D,), jnp.float32),
            pltpu.VMEM((H * CH * LN,), jnp.float32),
            pltpu.VMEM((SLAB,), jnp.float32),
            pltpu.VMEM((2 * LN,), jnp.float32),
            pltpu.SemaphoreType.DMA,
            pltpu.SemaphoreType.DMA,
        ],
    )(x_flat, w2, w1, b16)

    parts = pl.kernel(
        _sc_accum,
        out_type=jax.ShapeDtypeStruct((NW, D), jnp.float32),
        mesh=_mesh(),
        compiler_params=_SC_PARAMS,
        scratch_types=[
            pltpu.VMEM((SLAB,), jnp.float32),
            pltpu.VMEM((2 * NW * LN,), jnp.float32),
            pltpu.VMEM((D,), jnp.float32),
            pltpu.VMEM((CH * D,), jnp.float32),
            pltpu.VMEM((CH * D,), jnp.float32),
            pltpu.SemaphoreType.DMA,
            pltpu.SemaphoreType.DMA,
        ],
    )(x_flat, logits, stats)

    parts = parts.reshape(NW * D)
    out = pl.kernel(
        _sc_merge,
        out_type=jax.ShapeDtypeStruct((D,), jnp.float32),
        mesh=_mesh(),
        compiler_params=_SC_PARAMS,
        scratch_types=[
            pltpu.VMEM((NW * D,), jnp.float32),
            pltpu.VMEM((LN,), jnp.float32),
        ],
    )(parts)

    return out


def kernel(attention_mx, W, b):
    return _run(attention_mx, W, b)


# speedup vs baseline: 12.3238x; 12.2119x over previous
import functools

import jax
import jax.numpy as jnp
from jax.experimental import pallas as pl
from jax.experimental.pallas import tpu as pltpu

N = 10000
D = 512
H = 8
BLK = 2000


def _body(x_ref, w_ref, b_ref, o_ref, c_ref, m_ref, s_ref, acc_ref):
    i = pl.program_id(0)

    x = x_ref[...]
    w2 = w_ref[:, D:]

    @pl.when(i == 0)
    def _init():
        x0 = x_ref[0:1, :]
        w1 = w_ref[:, :D]
        c_ref[...] = jax.lax.dot_general(
            x0, w1, (((1,), (1,)), ((), ())),
            preferred_element_type=jnp.float32) + b_ref[...]
        s_ref[...] = jnp.zeros_like(s_ref)
        acc_ref[...] = jnp.zeros_like(acc_ref)

    l = jax.lax.dot_general(
        x, w2, (((1,), (1,)), ((), ())),
        preferred_element_type=jnp.float32) + c_ref[...]
    l = jnp.where(l >= 0, l, 0.01 * l)

    @pl.when(i == 0)
    def _setshift():
        m_ref[...] = jnp.max(l, axis=0, keepdims=True)

    p = jnp.exp(l - m_ref[...])
    s_ref[...] = s_ref[...] + jnp.sum(p, axis=0, keepdims=True)
    acc_ref[...] = acc_ref[...] + jax.lax.dot_general(
        p, x, (((0,), (0,)), ((), ())),
        preferred_element_type=jnp.float32)

    @pl.when(i == pl.num_programs(0) - 1)
    def _fin():
        head_avg = jnp.sum(acc_ref[...] / s_ref[...].T, axis=0,
                           keepdims=True) / H
        o_ref[...] = jnp.maximum(head_avg, 0.0)


@jax.jit
def _run(attention_mx, W, b):
    out = pl.pallas_call(
        _body,
        grid=(N // BLK,),
        in_specs=[
            pl.BlockSpec((BLK, D), lambda i: (i, 0)),
            pl.BlockSpec((H, 2 * D), lambda i: (0, 0)),
            pl.BlockSpec((1, H), lambda i: (0, 0)),
        ],
        out_specs=pl.BlockSpec((1, D), lambda i: (0, 0)),
        out_shape=jax.ShapeDtypeStruct((1, D), jnp.float32),
        scratch_shapes=[
            pltpu.VMEM((1, H), jnp.float32),
            pltpu.VMEM((1, H), jnp.float32),
            pltpu.VMEM((1, H), jnp.float32),
            pltpu.VMEM((H, D), jnp.float32),
        ],
    )(attention_mx, W, b.reshape(1, H))
    return out.reshape(D)


def kernel(attention_mx, W, b):
    return _run(attention_mx, W, b)


# speedup vs baseline: 12.9327x; 1.0494x over previous
import functools

import jax
import jax.numpy as jnp
from jax.experimental import pallas as pl
from jax.experimental.pallas import tpu as pltpu

N = 10000
D = 512
H = 8
BLK = 2000


def _body(x_ref, w_ref, b_ref, o_ref, c_ref, m_ref, s_ref, acc_ref):
    i = pl.program_id(0)

    @pl.when(i == 0)
    def _init():
        x0 = x_ref[0:1, :]
        w1 = w_ref[:, :D]
        c_ref[...] = jax.lax.dot_general(
            x0, w1, (((1,), (1,)), ((), ())),
            preferred_element_type=jnp.float32) + b_ref[...]
        m_ref[...] = jnp.full_like(m_ref, -jnp.inf)
        s_ref[...] = jnp.zeros_like(s_ref)
        acc_ref[...] = jnp.zeros_like(acc_ref)

    x = x_ref[...]
    w2 = w_ref[:, D:]
    l = jax.lax.dot_general(
        x, w2, (((1,), (1,)), ((), ())),
        preferred_element_type=jnp.float32) + c_ref[...]
    l = jnp.where(l >= 0, l, 0.01 * l)

    m_old = m_ref[...]
    m_new = jnp.maximum(m_old, jnp.max(l, axis=0, keepdims=True))
    p = jnp.exp(l - m_new)
    r = jnp.exp(m_old - m_new)
    m_ref[...] = m_new
    s_ref[...] = s_ref[...] * r + jnp.sum(p, axis=0, keepdims=True)
    acc_ref[...] = acc_ref[...] * r.T + jax.lax.dot_general(
        p, x, (((0,), (0,)), ((), ())),
        preferred_element_type=jnp.float32)

    @pl.when(i == pl.num_programs(0) - 1)
    def _fin():
        head_avg = jnp.sum(acc_ref[...] / s_ref[...].T, axis=0,
                           keepdims=True) / H
        o_ref[...] = jnp.maximum(head_avg, 0.0)


@jax.jit
def _run(attention_mx, W, b):
    out = pl.pallas_call(
        _body,
        grid=(N // BLK,),
        in_specs=[
            pl.BlockSpec((BLK, D), lambda i: (i, 0)),
            pl.BlockSpec((H, 2 * D), lambda i: (0, 0)),
            pl.BlockSpec((1, H), lambda i: (0, 0)),
        ],
        out_specs=pl.BlockSpec((1, D), lambda i: (0, 0)),
        out_shape=jax.ShapeDtypeStruct((1, D), jnp.float32),
        scratch_shapes=[
            pltpu.VMEM((1, H), jnp.float32),
            pltpu.VMEM((1, H), jnp.float32),
            pltpu.VMEM((1, H), jnp.float32),
            pltpu.VMEM((H, D), jnp.float32),
        ],
    )(attention_mx, W, b.reshape(1, H))
    return out.reshape(D)


def kernel(attention_mx, W, b):
    return _run(attention_mx, W, b)


# speedup vs baseline: 14.6161x; 1.1302x over previous
import functools

import jax
import jax.numpy as jnp
from jax.experimental import pallas as pl
from jax.experimental.pallas import tpu as pltpu

N = 10000
D = 512
H = 8
BLK = 1000


def _body(x1_ref, x2_ref, w_ref, b_ref, o_ref, c_ref, m_ref, s_ref, acc_ref):
    i = pl.program_id(0)

    @pl.when(i == 0)
    def _init():
        x0 = x1_ref[0:1, :]
        w1 = w_ref[:, :D]
        c_ref[...] = jax.lax.dot_general(
            x0, w1, (((1,), (1,)), ((), ())),
            preferred_element_type=jnp.float32) + b_ref[...]
        m_ref[...] = jnp.full_like(m_ref, -jnp.inf)
        s_ref[...] = jnp.zeros_like(s_ref)
        acc_ref[...] = jnp.zeros_like(acc_ref)

    x1 = x1_ref[...]
    x2 = x2_ref[...]
    w2 = w_ref[:, D:]
    l1 = jax.lax.dot_general(
        x1, w2, (((1,), (1,)), ((), ())),
        preferred_element_type=jnp.float32) + c_ref[...]
    l2 = jax.lax.dot_general(
        x2, w2, (((1,), (1,)), ((), ())),
        preferred_element_type=jnp.float32) + c_ref[...]
    l1 = jnp.where(l1 >= 0, l1, 0.01 * l1)
    l2 = jnp.where(l2 >= 0, l2, 0.01 * l2)

    m_old = m_ref[...]
    m_blk = jnp.maximum(jnp.max(l1, axis=0, keepdims=True),
                        jnp.max(l2, axis=0, keepdims=True))
    m_new = jnp.maximum(m_old, m_blk)
    p1 = jnp.exp(l1 - m_new)
    p2 = jnp.exp(l2 - m_new)
    r = jnp.exp(m_old - m_new)
    m_ref[...] = m_new
    s_ref[...] = (s_ref[...] * r
                  + jnp.sum(p1, axis=0, keepdims=True)
                  + jnp.sum(p2, axis=0, keepdims=True))
    acc_ref[...] = (acc_ref[...] * r.T
                    + jax.lax.dot_general(
                        p1, x1, (((0,), (0,)), ((), ())),
                        preferred_element_type=jnp.float32)
                    + jax.lax.dot_general(
                        p2, x2, (((0,), (0,)), ((), ())),
                        preferred_element_type=jnp.float32))

    @pl.when(i == pl.num_programs(0) - 1)
    def _fin():
        head_avg = jnp.sum(acc_ref[...] / s_ref[...].T, axis=0,
                           keepdims=True) / H
        o_ref[...] = jnp.maximum(head_avg, 0.0)


@jax.jit
def _run(attention_mx, W, b):
    out = pl.pallas_call(
        _body,
        grid=(N // (2 * BLK),),
        in_specs=[
            pl.BlockSpec((BLK, D), lambda i: (2 * i, 0)),
            pl.BlockSpec((BLK, D), lambda i: (2 * i + 1, 0)),
            pl.BlockSpec((H, 2 * D), lambda i: (0, 0)),
            pl.BlockSpec((1, H), lambda i: (0, 0)),
        ],
        out_specs=pl.BlockSpec((1, D), lambda i: (0, 0)),
        out_shape=jax.ShapeDtypeStruct((1, D), jnp.float32),
        scratch_shapes=[
            pltpu.VMEM((1, H), jnp.float32),
            pltpu.VMEM((1, H), jnp.float32),
            pltpu.VMEM((1, H), jnp.float32),
            pltpu.VMEM((H, D), jnp.float32),
        ],
    )(attention_mx, attention_mx, W, b.reshape(1, H))
    return out.reshape(D)


def kernel(attention_mx, W, b):
    return _run(attention_mx, W, b)
